# Initial kernel scaffold; baseline (speedup 1.0000x reference)
import functools
import math

import jax
import jax.numpy as jnp
import numpy as np
from jax.experimental import pallas as pl

EMB = 128
VOCAB = 6
NUM_CATS = 170
MAX_LEN = 5000


def _make_pe(d_model, max_len=MAX_LEN):
    position = np.arange(0, max_len, dtype=np.float32)[:, None]
    div_term = np.exp(np.arange(0, d_model, 2).astype(np.float32) * (-math.log(10000.0) / d_model))
    pe = np.zeros((max_len, d_model), dtype=np.float32)
    pe[:, 0::2] = np.sin(position * div_term)
    pe[:, 1::2] = np.cos(position * div_term)
    return pe


_PE = _make_pe(EMB)


def _embed_kernel(seqs_ref, cats_ref, vocab_ref, cat_ref, pe_ref, gb_ref, out_ref):
    seqs = seqs_ref[...]
    cats = cats_ref[...]

    cat_ids = jax.lax.broadcasted_iota(jnp.int32, (cats.shape[0], cat_ref.shape[0]), 1)
    onehot = (cats == cat_ids).astype(jnp.float32)
    cat_emb = jnp.dot(onehot, cat_ref[...], preferred_element_type=jnp.float32)

    x = jnp.broadcast_to(cat_emb[:, None, :] + pe_ref[...][None, :, :],
                         (seqs.shape[0], seqs.shape[1], EMB))
    for v in range(VOCAB):
        row = vocab_ref[v, :]
        x = x + jnp.where((seqs == v)[:, :, None], row[None, None, :], 0.0)

    mean = jnp.mean(x, axis=-1, keepdims=True)
    xc = x - mean
    var = jnp.mean(xc * xc, axis=-1, keepdims=True)
    normed = xc * jax.lax.rsqrt(var + 1e-12)
    gamma = gb_ref[0, :][None, None, :]
    beta = gb_ref[1, :][None, None, :]
    out_ref[...] = normed * gamma + beta


@functools.partial(jax.jit, static_argnames=())
def kernel(seqs, cats, vocab_table, cat_table, gamma, beta):
    B, L = seqs.shape
    BB = 128
    pe = jnp.asarray(_PE[:L])
    gb = jnp.stack([gamma, beta], axis=0)
    cats2d = cats.reshape(B, 1).astype(jnp.int32)
    seqs = seqs.astype(jnp.int32)

    grid = (B // BB,)
    out = pl.pallas_call(
        _embed_kernel,
        grid=grid,
        in_specs=[
            pl.BlockSpec((BB, L), lambda i: (i, 0)),
            pl.BlockSpec((BB, 1), lambda i: (i, 0)),
            pl.BlockSpec((VOCAB, EMB), lambda i: (0, 0)),
            pl.BlockSpec((NUM_CATS, EMB), lambda i: (0, 0)),
            pl.BlockSpec((L, EMB), lambda i: (0, 0)),
            pl.BlockSpec((2, EMB), lambda i: (0, 0)),
        ],
        out_specs=pl.BlockSpec((BB, L, EMB), lambda i: (i, 0, 0)),
        out_shape=jax.ShapeDtypeStruct((B, L, EMB), jnp.float32),
    )(seqs, cats2d, vocab_table, cat_table, pe, gb)
    return out


# speedup vs baseline: 3.3423x; 3.3423x over previous
import functools
import math

import jax
import jax.numpy as jnp
import numpy as np
from jax.experimental import pallas as pl

EMB = 128
VOCAB = 6
NUM_CATS = 170
MAX_LEN = 5000


def _make_pe(d_model, max_len=MAX_LEN):
    position = np.arange(0, max_len, dtype=np.float32)[:, None]
    div_term = np.exp(np.arange(0, d_model, 2).astype(np.float32) * (-math.log(10000.0) / d_model))
    pe = np.zeros((max_len, d_model), dtype=np.float32)
    pe[:, 0::2] = np.sin(position * div_term)
    pe[:, 1::2] = np.cos(position * div_term)
    return pe


_PE = _make_pe(EMB)


def _cat_kernel(cats_ref, cat_ref, out_ref):
    cats = cats_ref[...]
    ids = jax.lax.broadcasted_iota(jnp.int32, (cats.shape[0], cat_ref.shape[0]), 1)
    onehot = (cats == ids).astype(jnp.float32)
    out_ref[...] = jnp.dot(onehot, cat_ref[...], preferred_element_type=jnp.float32)


def _embed_kernel(seqs_ref, catemb_ref, vocab_ref, pe_ref, gb_ref, out_ref):
    seqs = seqs_ref[...]
    x = catemb_ref[...] + pe_ref[...]
    for v in range(VOCAB):
        row = vocab_ref[v, :][None, None, :]
        x = x + jnp.where(seqs == v, row, 0.0)

    mean = jnp.mean(x, axis=-1, keepdims=True)
    xc = x - mean
    var = jnp.mean(xc * xc, axis=-1, keepdims=True)
    normed = xc * jax.lax.rsqrt(var + 1e-12)
    gamma = gb_ref[0, :][None, None, :]
    beta = gb_ref[1, :][None, None, :]
    out_ref[...] = normed * gamma + beta


@functools.partial(jax.jit, static_argnames=())
def kernel(seqs, cats, vocab_table, cat_table, gamma, beta):
    B, L = seqs.shape
    BB = 64
    pe3 = jnp.asarray(_PE[:L])[None, :, :]
    gb = jnp.stack([gamma, beta], axis=0)
    cats2d = cats.reshape(B, 1).astype(jnp.int32)
    seqs3 = seqs.astype(jnp.int32)[:, :, None]

    cat_emb = pl.pallas_call(
        _cat_kernel,
        out_shape=jax.ShapeDtypeStruct((B, EMB), jnp.float32),
    )(cats2d, cat_table)
    cat_emb3 = cat_emb[:, None, :]

    grid = (B // BB,)
    out = pl.pallas_call(
        _embed_kernel,
        grid=grid,
        in_specs=[
            pl.BlockSpec((BB, L, 1), lambda i: (i, 0, 0)),
            pl.BlockSpec((BB, 1, EMB), lambda i: (i, 0, 0)),
            pl.BlockSpec((VOCAB, EMB), lambda i: (0, 0)),
            pl.BlockSpec((1, L, EMB), lambda i: (0, 0, 0)),
            pl.BlockSpec((2, EMB), lambda i: (0, 0)),
        ],
        out_specs=pl.BlockSpec((BB, L, EMB), lambda i: (i, 0, 0)),
        out_shape=jax.ShapeDtypeStruct((B, L, EMB), jnp.float32),
    )(seqs3, cat_emb3, vocab_table, pe3, gb)
    return out
